# baseline (speedup 1.0000x reference)
import functools

import jax
import jax.numpy as jnp
from jax import lax
from jax.experimental import pallas as pl
from jax.experimental.pallas import tpu as pltpu
from jax.experimental.pallas import tpu_sc as plsc

_L = 16
_K = 128
_NBUF = 3
_NW = 32


def _matmul(x, filters):
    n, f = x.shape
    out = filters.shape[1]
    blk = 400
    assert n % blk == 0

    def body(x_ref, w_ref, y_ref):
        y_ref[...] = jnp.dot(x_ref[...], w_ref[...],
                             preferred_element_type=jnp.float32)

    return pl.pallas_call(
        body,
        grid=(n // blk,),
        in_specs=[
            pl.BlockSpec((blk, f), lambda i: (i, 0)),
            pl.BlockSpec((f, out), lambda i: (0, 0)),
        ],
        out_specs=pl.BlockSpec((blk, out), lambda i: (i, 0)),
        out_shape=jax.ShapeDtypeStruct((n, out), jnp.float32),
    )(x, filters)


def _make_sc_spmv(n, out, n_edges):
    small = (n // _NW) // 8 * 8
    nbig = (n - _NW * small) // 8
    big = small + 8
    acc_rows = big + _L
    dummy = big + 8
    nvec = out // _L
    mesh = plsc.VectorSubcoreMesh(core_axis_name="c", subcore_axis_name="s")

    @functools.partial(
        pl.kernel,
        out_type=jax.ShapeDtypeStruct((n, out), jnp.float32),
        mesh=mesh,
        scratch_types=[
            [pltpu.VMEM((_K,), jnp.int32)] * _NBUF,
            [pltpu.VMEM((_K,), jnp.int32)] * _NBUF,
            [pltpu.VMEM((_K,), jnp.float32)] * _NBUF,
            [pltpu.VMEM((_K, out), jnp.float32)] * _NBUF,
            pltpu.VMEM((acc_rows, out), jnp.float32),
            pltpu.VMEM((_L,), jnp.int32),
            [pltpu.SemaphoreType.DMA] * _NBUF,
        ],
    )
    def sc_kernel(y_hbm, cols_hbm, rows_hbm, vals_hbm, out_hbm,
                  cidx, ridx, vbuf, gbuf, accv, probe, sem):
        cid = lax.axis_index("c")
        sid = lax.axis_index("s")
        w = sid * 2 + cid
        iota = lax.iota(jnp.int32, _L)

        r_lo = jnp.where(w < nbig, big * w,
                         big * nbig + small * (w - nbig))
        r_hi = r_lo + jnp.where(w < nbig, big, small)

        zeros16 = jnp.zeros((_L,), jnp.float32)

        def zrow(r, carry):
            for j in range(nvec):
                accv[r, pl.ds(j * _L, _L)] = zeros16
            return carry

        lax.fori_loop(0, acc_rows, zrow, 0)

        def bsearch(target):
            def bs_body(i, state):
                lo, hi = state
                mid = (lo + hi) // 2
                m0 = pl.multiple_of((mid // 8) * 8, 8)
                pltpu.sync_copy(rows_hbm.at[pl.ds(m0, _L)], probe)
                v = probe[pl.ds(0, _L)]
                lane = mid - m0
                val = v[0]
                for l in range(1, 8):
                    val = jnp.where(lane == l, v[l], val)
                go_right = val < target
                done = lo >= hi
                return (jnp.where(done, lo,
                                  jnp.where(go_right, mid + 1, lo)),
                        jnp.where(done, hi,
                                  jnp.where(go_right, hi, mid)))

            return lax.fori_loop(0, max(n_edges, 2).bit_length(),
                                 bs_body, (0, n_edges))[0]

        s = bsearch(r_lo)
        e = bsearch(r_hi)
        s0 = (s // 8) * 8
        nb = jnp.maximum((e - s0 + _K - 1) // _K, 0)

        def load_mask_gather(k, b):
            base = pl.multiple_of(s0 + k * _K, 8)
            pltpu.sync_copy(cols_hbm.at[pl.ds(base, _K)], cidx[b])
            pltpu.sync_copy(rows_hbm.at[pl.ds(base, _K)], ridx[b])
            pltpu.sync_copy(vals_hbm.at[pl.ds(base, _K)], vbuf[b])
            for j in range(_K // _L):
                gid = base + j * _L + iota
                r16 = ridx[b][pl.ds(j * _L, _L)]
                valid = (gid >= s) & (gid < e)
                ridx[b][pl.ds(j * _L, _L)] = jnp.where(valid, r16 - r_lo,
                                                       dummy)
            pltpu.async_copy(y_hbm.at[cidx[b]], gbuf[b], sem[b])

        def wait_scale_accum(b):
            pltpu.make_async_copy(y_hbm.at[cidx[b]], gbuf[b], sem[b]).wait()

            def vgroup(g, carry):
                g16 = pl.multiple_of(g * _L, _L)
                vvec = vbuf[b][pl.ds(g16, _L)]
                rvec = ridx[b][pl.ds(g16, _L)]
                for i in range(_L):
                    vv = jnp.full((_L,), vvec[i])
                    r_loc = rvec[i]
                    row = g16 + i
                    for j in range(nvec):
                        sl = pl.ds(j * _L, _L)
                        plsc.addupdate(accv.at[r_loc, sl],
                                       gbuf[b][row, sl] * vv)
                return carry

            lax.fori_loop(0, _K // _L, vgroup, 0)

        for b in range(_NBUF):
            load_mask_gather(b, b)

        def outer(g, carry):
            for b in range(_NBUF):
                wait_scale_accum(b)
                load_mask_gather(g * _NBUF + b + _NBUF, b)
            return carry

        lax.fori_loop(0, (nb + _NBUF - 1) // _NBUF, outer, 0)
        for b in range(_NBUF):
            wait_scale_accum(b)

        glob0 = pl.multiple_of(r_lo, 8)

        @pl.when(w < nbig)
        def _():
            pltpu.sync_copy(accv.at[pl.ds(0, big)],
                            out_hbm.at[pl.ds(glob0, big)])

        @pl.when(w >= nbig)
        def _():
            pltpu.sync_copy(accv.at[pl.ds(0, small)],
                            out_hbm.at[pl.ds(glob0, small)])

    return sc_kernel


def kernel(x, filters, t_vals, t_rows, t_cols):
    n, f = x.shape
    out = filters.shape[1]
    e = t_rows.shape[0]

    y = _matmul(x, filters)

    e_pad = (e + 7) // 8 * 8 + 8 * _K
    pad = e_pad - e
    cols_p = jnp.concatenate([t_cols, jnp.zeros((pad,), jnp.int32)])
    rows_p = jnp.concatenate([t_rows, jnp.full((pad,), n - 1, jnp.int32)])
    vals_p = jnp.concatenate([t_vals, jnp.zeros((pad,), jnp.float32)])

    return _make_sc_spmv(n, out, e)(y, cols_p, rows_p, vals_p)


if __name__ == "__main__":
    import numpy as np
    import reference as _r

    inputs = _r.setup_inputs(0)
    got = kernel(inputs["x"], inputs["filters"], inputs["t_vals"],
                 inputs["t_rows"], inputs["t_cols"])
    want = _r.reference(inputs["x"], inputs["filters"], inputs["t_vals"],
                        inputs["t_rows"], inputs["t_cols"])
    err = float(np.mean((np.asarray(got) - np.asarray(want)) ** 2)
                / np.mean(np.asarray(want) ** 2))
    print("resid var ratio:", err)


# speedup vs baseline: 2.0491x; 2.0491x over previous
import functools

import jax
import jax.numpy as jnp
from jax import lax
from jax.experimental import pallas as pl
from jax.experimental.pallas import tpu as pltpu
from jax.experimental.pallas import tpu_sc as plsc

_L = 16
_K = 128
_SB = 4


def _matmul(x, filters):
    n, f = x.shape
    out = filters.shape[1]
    blk = 400
    assert n % blk == 0

    def body(x_ref, w_ref, y_ref):
        y_ref[...] = jnp.dot(x_ref[...], w_ref[...],
                             preferred_element_type=jnp.float32)

    return pl.pallas_call(
        body,
        grid=(n // blk,),
        in_specs=[
            pl.BlockSpec((blk, f), lambda i: (i, 0)),
            pl.BlockSpec((f, out), lambda i: (0, 0)),
        ],
        out_specs=pl.BlockSpec((blk, out), lambda i: (i, 0)),
        out_shape=jax.ShapeDtypeStruct((n, out), jnp.float32),
    )(x, filters)


def _make_sc_spmv(n, out, n_edges):
    half = n // 2
    fl = (half // 16) // 8 * 8
    tail = half - 16 * fl
    acc_rows = (half // _K + 1) * _K
    nzc = acc_rows // _K
    dummy = half + 16
    nvec = out // _L
    mesh = plsc.VectorSubcoreMesh(core_axis_name="c", subcore_axis_name="s")

    @functools.partial(
        pl.kernel,
        out_type=jax.ShapeDtypeStruct((n, out), jnp.float32),
        mesh=mesh,
        scratch_types=[
            [pltpu.VMEM((_SB * _K,), jnp.int32)] * 2,
            [pltpu.VMEM((_SB * _K,), jnp.int32)] * 2,
            [pltpu.VMEM((_SB, _K), jnp.int32)] * 2,
            [pltpu.VMEM((_SB * _K,), jnp.float32)] * 2,
            pltpu.VMEM((_K,), jnp.int32),
            [pltpu.VMEM((_K, out), jnp.float32)] * _SB,
            pltpu.VMEM((_K, out), jnp.float32),
            pltpu.VMEM((_L,), jnp.int32),
            pltpu.VMEM_SHARED((acc_rows, out), jnp.float32),
            [pltpu.SemaphoreType.DMA] * 2,
            [pltpu.SemaphoreType.DMA] * _SB,
        ],
    )
    def sc_kernel(y_hbm, cols_hbm, rows_hbm, vals_hbm, out_hbm,
                  cbig, rraw, rbig, vbig, zidx, gbuf, fbuf, probe, acc,
                  sem_i, sem_g):
        cid = lax.axis_index("c")
        sid = lax.axis_index("s")
        row_base = cid * half
        iota = lax.iota(jnp.int32, _L)

        def fill_zidx(base):
            for j in range(_K // _L):
                zidx[pl.ds(j * _L, _L)] = base + j * _L + iota

        zeros16 = jnp.zeros((_L,), jnp.float32)

        def zrow(r, carry):
            for j in range(nvec):
                fbuf[r, pl.ds(j * _L, _L)] = zeros16
            return carry

        lax.fori_loop(0, _K, zrow, 0)
        for c in range((nzc + 15) // 16):
            chunk = sid + c * 16

            @pl.when(chunk < nzc)
            def _():
                fill_zidx(chunk * _K)
                pltpu.sync_copy(fbuf, acc.at[zidx])

        def bs_body(i, state):
            lo, hi = state
            mid = (lo + hi) // 2
            m0 = pl.multiple_of((mid // 8) * 8, 8)
            pltpu.sync_copy(rows_hbm.at[pl.ds(m0, _L)], probe)
            v = probe[pl.ds(0, _L)]
            lane = mid % 8
            val = v[0]
            for l in range(1, 8):
                val = jnp.where(lane == l, v[l], val)
            go_right = val < half
            done = lo >= hi
            return (jnp.where(done, lo, jnp.where(go_right, mid + 1, lo)),
                    jnp.where(done, hi, jnp.where(go_right, hi, mid)))

        b0, _ = lax.fori_loop(0, max(n_edges, 2).bit_length(),
                              bs_body, (0, n_edges))
        plsc.subcore_barrier()

        lo = jnp.where(cid == 0, 0, b0)
        hi = jnp.where(cid == 0, b0, n_edges)
        total = hi - lo
        q = total // 16
        rem = total % 16
        s = lo + sid * q + jnp.minimum(sid, rem)
        e = s + q + jnp.where(sid < rem, 1, 0)
        sbe = _SB * _K
        s0 = s // sbe * sbe
        nsb = jnp.maximum((e - s0 + sbe - 1) // sbe, 0)
        npair = (nsb + 1) // 2

        def idx_load(p, sb, sync):
            base = pl.multiple_of(s0 + sb * sbe, 8)
            copy = pltpu.sync_copy if sync else (
                lambda src, dst: pltpu.async_copy(src, dst, sem_i[p]))
            copy(cols_hbm.at[pl.ds(base, sbe)], cbig[p])
            copy(rows_hbm.at[pl.ds(base, sbe)], rraw[p])
            copy(vals_hbm.at[pl.ds(base, sbe)], vbig[p])

        def idx_wait(p, sb):
            base = pl.multiple_of(s0 + sb * sbe, 8)
            for src, dst in ((cols_hbm, cbig), (rows_hbm, rraw),
                             (vals_hbm, vbig)):
                pltpu.make_async_copy(src.at[pl.ds(base, sbe)], dst[p],
                                      sem_i[p]).wait()

        def gather(p, j):
            pltpu.async_copy(y_hbm.at[cbig[p].at[pl.ds(j * _K, _K)]],
                             gbuf[j], sem_g[j])

        def process(p, sb):
            idx_wait(1 - p, sb + 1)
            base = s0 + sb * sbe
            for j in range(_SB):
                for g in range(_K // _L):
                    gid = base + j * _K + g * _L + iota
                    r16 = rraw[p][pl.ds(j * _K + g * _L, _L)]
                    valid = (gid >= s) & (gid < e)
                    rbig[p][j, pl.ds(g * _L, _L)] = jnp.where(
                        valid, r16 - row_base, dummy)
            for j in range(_SB):
                pltpu.make_async_copy(y_hbm.at[cbig[p].at[pl.ds(j * _K,
                                                                _K)]],
                                      gbuf[j], sem_g[j]).wait()

                def vgroup(g, carry, j=j):
                    g16 = pl.multiple_of(g * _L, _L)
                    vvec = vbig[p][pl.ds(j * _K + g16, _L)]
                    for i in range(_L):
                        vv = jnp.full((_L,), vvec[i])
                        row = g16 + i
                        for q_ in range(nvec):
                            sl = pl.ds(q_ * _L, _L)
                            gbuf[j][row, sl] = gbuf[j][row, sl] * vv
                    return carry

                lax.fori_loop(0, _K // _L, vgroup, 0)
                pltpu.sync_copy(gbuf[j], acc.at[rbig[p].at[j]], add=True)
                gather(1 - p, j)
            idx_load(p, sb + 2, sync=False)

        idx_load(0, 0, sync=True)
        idx_load(1, 1, sync=False)
        for j in range(_SB):
            gather(0, j)

        def pair(t, carry):
            process(0, 2 * t)
            process(1, 2 * t + 1)
            return carry

        lax.fori_loop(0, npair, pair, 0)

        for j in range(_SB):
            pltpu.make_async_copy(y_hbm.at[cbig[0].at[pl.ds(j * _K, _K)]],
                                  gbuf[j], sem_g[j]).wait()
        idx_wait(1, 2 * npair + 1)
        plsc.subcore_barrier()

        def flush_chunk(local0, cnt):
            fill_zidx(local0)
            pltpu.sync_copy(acc.at[zidx], fbuf)
            glob0 = pl.multiple_of(row_base + local0, 8)
            pltpu.sync_copy(fbuf.at[pl.ds(0, cnt)],
                            out_hbm.at[pl.ds(glob0, cnt)])

        off = 0
        while off < fl:
            cnt = min(_K, fl - off)
            flush_chunk(sid * fl + off, cnt)
            off += cnt

        @pl.when(sid == 0)
        def _():
            flush_chunk(16 * fl, tail)

    return sc_kernel


def kernel(x, filters, t_vals, t_rows, t_cols):
    n, f = x.shape
    out = filters.shape[1]
    e = t_rows.shape[0]

    y = _matmul(x, filters)

    sbe = _SB * _K
    e_pad = (e + 4 * sbe - 1) // sbe * sbe + 4 * sbe
    pad = e_pad - e
    cols_p = jnp.concatenate([t_cols, jnp.zeros((pad,), jnp.int32)])
    rows_p = jnp.concatenate([t_rows, jnp.full((pad,), n - 1, jnp.int32)])
    vals_p = jnp.concatenate([t_vals, jnp.zeros((pad,), jnp.float32)])

    return _make_sc_spmv(n, out, e)(y, cols_p, rows_p, vals_p)


if __name__ == "__main__":
    import numpy as np
    import reference as _r

    inputs = _r.setup_inputs(0)
    got = kernel(inputs["x"], inputs["filters"], inputs["t_vals"],
                 inputs["t_rows"], inputs["t_cols"])
    want = _r.reference(inputs["x"], inputs["filters"], inputs["t_vals"],
                        inputs["t_rows"], inputs["t_cols"])
    err = float(np.mean((np.asarray(got) - np.asarray(want)) ** 2)
                / np.mean(np.asarray(want) ** 2))
    print("resid var ratio:", err)
